# baseline (speedup 1.0000x reference)
import jax
import jax.numpy as jnp
from jax import lax
from jax.experimental import pallas as pl
from jax.experimental.pallas import tpu as pltpu

_B, _C, _O, _H, _W = 4, 96, 96, 224, 224
_E, _K, _PS, _NG = 8, 2, 8, 8
_R = 6
_HW = _H * _W
_THRESH = 0.01
_GS = _O // _NG
_T = 7168
_NT = _HW // _T
_TR = _T // _W
_PW = _W // _PS
_PR = _TR // _PS
_PPC = _PR * _PW


def _silu(v):
    return v * jax.nn.sigmoid(v)


def _fused_kernel(x_ref, pmat_ref, dw_ref, pw_ref, fc_ref, fcb_ref,
                  shw_ref, shsb_ref, expw_ref, expsb_ref, out_ref,
                  xc, gacc, pool_s, m_s, aff_s):
    p = pl.program_id(1)

    @pl.when(p < _NT)
    def _phase1():
        xt = x_ref[0]
        xc[p] = xt
        xtb = xt.astype(jnp.bfloat16)
        g = lax.dot_general(xtb, xtb, (((1,), (1,)), ((), ())),
                            preferred_element_type=jnp.float32)

        @pl.when(p == 0)
        def _():
            gacc[...] = g

        @pl.when(p != 0)
        def _():
            gacc[...] += g

        pool_s[p] = jnp.dot(xt, pmat_ref[...],
                            preferred_element_type=jnp.float32)

    @pl.when(p == _NT)
    def _boundary():
        xm = jnp.concatenate([pool_s[k] for k in range(_NT)], axis=1)
        z32 = jnp.zeros((_C, 32), jnp.float32)
        xbig = jnp.concatenate([z32, xm, z32], axis=1)
        jcol = lax.broadcasted_iota(jnp.int32, (_C, _PW * _PW), 1) % _PW
        acc = jnp.zeros((_C, _PW * _PW), jnp.float32)
        for di in (-1, 0, 1):
            for dj in (-1, 0, 1):
                k9 = (di + 1) * 3 + (dj + 1)
                base = 32 + _PW * di + dj
                term = xbig[:, base:base + _PW * _PW] * dw_ref[:, k9:k9 + 1]
                if dj == -1:
                    term = jnp.where(jcol == 0, 0.0, term)
                elif dj == 1:
                    term = jnp.where(jcol == _PW - 1, 0.0, term)
                acc = acc + term
        xd = _silu(acc)
        xr = _silu(jnp.dot(pw_ref[...], xd, preferred_element_type=jnp.float32))
        gap = jnp.mean(xr, axis=1, keepdims=True)
        logits = jnp.dot(fc_ref[...], gap,
                         preferred_element_type=jnp.float32) + fcb_ref[...]
        io = lax.broadcasted_iota(jnp.int32, (_E, 1), 0)
        m1 = jnp.max(logits)
        i1 = jnp.min(jnp.where(logits == m1, io, _E))
        m2 = jnp.max(jnp.where(io == i1, -1e30, logits))
        i2 = jnp.min(jnp.where((logits == m2) & (io != i1), io, _E))
        e = jnp.exp(m2 - m1)
        w1 = 1.0 / (1.0 + e)
        w2 = e / (1.0 + e)
        w1 = jnp.where(w1 >= _THRESH, w1, 0.0)
        w2 = jnp.where(w2 >= _THRESH, w2, 0.0)

        g = gacc[...]
        s = jnp.sum(xm, axis=1, keepdims=True) * (_PS * _PS)
        gi = lax.broadcasted_iota(jnp.int32, (_O, _O), 0) // _GS
        gj = lax.broadcasted_iota(jnp.int32, (_O, _O), 1) // _GS
        pg = jnp.where(gi == gj, 1.0 / _GS, 0.0)
        for j, (idx, cw) in enumerate(((None, None), (i1, w1), (i2, w2))):
            if j == 0:
                wu = shw_ref[...]
                sb = shsb_ref[...]
                cval = jnp.float32(1.0)
            else:
                wu = expw_ref[idx]
                sb = expsb_ref[idx]
                cval = cw
            m = jnp.dot(wu, s, preferred_element_type=jnp.float32) / _HW
            t = jnp.dot(wu, g, preferred_element_type=jnp.float32)
            q = jnp.sum(t * wu, axis=1, keepdims=True) / _HW
            mu = jnp.dot(pg, m, preferred_element_type=jnp.float32)
            var = jnp.dot(pg, q, preferred_element_type=jnp.float32) - mu * mu
            rsig = lax.rsqrt(var + 1e-5)
            a = rsig * sb[:, 0:1]
            bv = sb[:, 1:2] - mu * a
            cc = jnp.zeros((_O, 1), jnp.float32) + cval
            m_s[j * _O:(j + 1) * _O] = (wu * a).astype(jnp.bfloat16)
            aff_s[j * _O:(j + 1) * _O] = jnp.concatenate(
                [bv, cc, jnp.zeros((_O, 6), jnp.float32)], axis=1)

    @pl.when(p >= _NT)
    def _phase2():
        t = p - _NT
        xt = xc[t]
        z = jnp.dot(m_s[...], xt.astype(jnp.bfloat16),
                    preferred_element_type=jnp.float32)
        acc = None
        for j in range(3):
            zj = z[_O * j:_O * (j + 1)]
            af = aff_s[j * _O:(j + 1) * _O]
            tj = zj + af[:, 0:1]
            oj = af[:, 1:2] * (tj * jax.nn.sigmoid(tj))
            acc = oj if acc is None else acc + oj
        out_ref[0] = acc


def kernel(x, router_dw_w, router_pw_w, router_fc_w, router_fc_b,
           shared_w, shared_gn_scale, shared_gn_bias,
           expert_w, expert_gn_scale, expert_gn_bias):
    x3 = x.reshape(_B, _C, _HW)

    ridx = jnp.arange(_T) // _W
    widx = jnp.arange(_T) % _W
    pcol = (ridx // _PS) * _PW + widx // _PS
    pmat = ((pcol[:, None] == jnp.arange(_PPC)[None, :])
            .astype(jnp.float32) / (_PS * _PS))

    dw9 = router_dw_w.reshape(_C, 9)
    pw8 = jnp.zeros((8, _C), jnp.float32).at[:_R].set(router_pw_w)
    fc8 = jnp.zeros((_E, 8), jnp.float32).at[:, :_R].set(router_fc_w)
    fcb = router_fc_b.reshape(_E, 1)
    expsb = jnp.stack([expert_gn_scale, expert_gn_bias], axis=-1)
    shsb = jnp.stack([shared_gn_scale, shared_gn_bias], axis=-1)

    cdims = lambda b, p: (b, 0, 0)
    out3 = pl.pallas_call(
        _fused_kernel,
        grid=(_B, 2 * _NT),
        in_specs=[
            pl.BlockSpec((1, _C, _T),
                         lambda b, p: (b, 0, jnp.minimum(p, _NT - 1))),
            pl.BlockSpec((_T, _PPC), lambda b, p: (0, 0)),
            pl.BlockSpec((_C, 9), lambda b, p: (0, 0)),
            pl.BlockSpec((8, _C), lambda b, p: (0, 0)),
            pl.BlockSpec((_E, 8), lambda b, p: (0, 0)),
            pl.BlockSpec((_E, 1), lambda b, p: (0, 0)),
            pl.BlockSpec((_O, _C), lambda b, p: (0, 0)),
            pl.BlockSpec((_O, 2), lambda b, p: (0, 0)),
            pl.BlockSpec((_E, _O, _C), lambda b, p: (0, 0, 0)),
            pl.BlockSpec((_E, _O, 2), lambda b, p: (0, 0, 0)),
        ],
        out_specs=pl.BlockSpec((1, _O, _T),
                               lambda b, p: (b, 0, jnp.maximum(p - _NT, 0))),
        out_shape=jax.ShapeDtypeStruct((_B, _O, _HW), jnp.float32),
        scratch_shapes=[
            pltpu.VMEM((_NT, _C, _T), jnp.float32),
            pltpu.VMEM((_C, _C), jnp.float32),
            pltpu.VMEM((_NT, _C, _PPC), jnp.float32),
            pltpu.VMEM((3 * _O, _C), jnp.bfloat16),
            pltpu.VMEM((3 * _O, 8), jnp.float32),
        ],
    )(x3, pmat, dw9, pw8, fc8, fcb, shared_w, shsb, expert_w, expsb)

    return out3.reshape(_B, _O, _H, _W)


# speedup vs baseline: 1.0218x; 1.0218x over previous
import jax
import jax.numpy as jnp
from jax import lax
from jax.experimental import pallas as pl
from jax.experimental.pallas import tpu as pltpu

_B, _C, _O, _H, _W = 4, 96, 96, 224, 224
_E, _K, _PS, _NG = 8, 2, 8, 8
_R = 6
_HW = _H * _W
_THRESH = 0.01
_GS = _O // _NG
_T = 7168
_NT = _HW // _T
_TR = _T // _W
_PW = _W // _PS
_PR = _TR // _PS
_PPC = _PR * _PW


def _silu(v):
    return v * jax.nn.sigmoid(v)


def _fused_kernel(x_ref, pmat_ref, dw_ref, pw_ref, fc_ref, fcb_ref,
                  shw_ref, shsb_ref, expw_ref, expsb_ref, out_ref,
                  xc, gacc, pool_s, m_s, aff_s):
    i = pl.program_id(0)
    t = pl.program_id(1)
    i2 = lax.rem(i, 2)
    oi2 = 1 - i2

    @pl.when(i < _B)
    def _phase1():
        xt = x_ref[0]
        xc[i2, t] = xt.astype(jnp.bfloat16)
        xtb = xt.astype(jnp.bfloat16)
        g = lax.dot_general(xtb, xtb, (((1,), (1,)), ((), ())),
                            preferred_element_type=jnp.float32)

        @pl.when(t == 0)
        def _():
            gacc[i2] = g

        @pl.when(t != 0)
        def _():
            gacc[i2] += g

        pool_s[i2, t] = jnp.dot(xt, pmat_ref[...],
                                preferred_element_type=jnp.float32)

    @pl.when((i > 0) & (t == 0))
    def _boundary():
        xm = jnp.concatenate([pool_s[oi2, k] for k in range(_NT)], axis=1)
        z32 = jnp.zeros((_C, 32), jnp.float32)
        xbig = jnp.concatenate([z32, xm, z32], axis=1)
        jcol = lax.broadcasted_iota(jnp.int32, (_C, _PW * _PW), 1) % _PW
        acc = jnp.zeros((_C, _PW * _PW), jnp.float32)
        for di in (-1, 0, 1):
            for dj in (-1, 0, 1):
                k9 = (di + 1) * 3 + (dj + 1)
                base = 32 + _PW * di + dj
                term = xbig[:, base:base + _PW * _PW] * dw_ref[:, k9:k9 + 1]
                if dj == -1:
                    term = jnp.where(jcol == 0, 0.0, term)
                elif dj == 1:
                    term = jnp.where(jcol == _PW - 1, 0.0, term)
                acc = acc + term
        xd = _silu(acc)
        xr = _silu(jnp.dot(pw_ref[...], xd, preferred_element_type=jnp.float32))
        gap = jnp.mean(xr, axis=1, keepdims=True)
        logits = jnp.dot(fc_ref[...], gap,
                         preferred_element_type=jnp.float32) + fcb_ref[...]
        io = lax.broadcasted_iota(jnp.int32, (_E, 1), 0)
        m1 = jnp.max(logits)
        i1 = jnp.min(jnp.where(logits == m1, io, _E))
        m2 = jnp.max(jnp.where(io == i1, -1e30, logits))
        i2 = jnp.min(jnp.where((logits == m2) & (io != i1), io, _E))
        e = jnp.exp(m2 - m1)
        w1 = 1.0 / (1.0 + e)
        w2 = e / (1.0 + e)
        w1 = jnp.where(w1 >= _THRESH, w1, 0.0)
        w2 = jnp.where(w2 >= _THRESH, w2, 0.0)

        g = gacc[oi2]
        s = jnp.sum(xm, axis=1, keepdims=True) * (_PS * _PS)
        gi = lax.broadcasted_iota(jnp.int32, (_O, _O), 0) // _GS
        gj = lax.broadcasted_iota(jnp.int32, (_O, _O), 1) // _GS
        pg = jnp.where(gi == gj, 1.0 / _GS, 0.0)
        for j, (idx, cw) in enumerate(((None, None), (i1, w1), (i2, w2))):
            if j == 0:
                wu = shw_ref[...]
                sb = shsb_ref[...]
                cval = jnp.float32(1.0)
            else:
                wu = expw_ref[idx]
                sb = expsb_ref[idx]
                cval = cw
            m = jnp.dot(wu, s, preferred_element_type=jnp.float32) / _HW
            t = jnp.dot(wu, g, preferred_element_type=jnp.float32)
            q = jnp.sum(t * wu, axis=1, keepdims=True) / _HW
            mu = jnp.dot(pg, m, preferred_element_type=jnp.float32)
            var = jnp.dot(pg, q, preferred_element_type=jnp.float32) - mu * mu
            rsig = lax.rsqrt(var + 1e-5)
            a = rsig * sb[:, 0:1]
            bv = sb[:, 1:2] - mu * a
            cc = jnp.zeros((_O, 1), jnp.float32) + cval
            m_s[j * _O:(j + 1) * _O] = (wu * a).astype(jnp.bfloat16)
            aff_s[j * _O:(j + 1) * _O] = jnp.concatenate(
                [bv, cc, jnp.zeros((_O, 6), jnp.float32)], axis=1)

    @pl.when(i > 0)
    def _phase2():
        xt = xc[oi2, t]
        z = jnp.dot(m_s[...], xt,
                    preferred_element_type=jnp.float32)
        acc = None
        for j in range(3):
            zj = z[_O * j:_O * (j + 1)]
            af = aff_s[j * _O:(j + 1) * _O]
            tj = zj + af[:, 0:1]
            oj = af[:, 1:2] * (tj * jax.nn.sigmoid(tj))
            acc = oj if acc is None else acc + oj
        out_ref[0] = acc


def kernel(x, router_dw_w, router_pw_w, router_fc_w, router_fc_b,
           shared_w, shared_gn_scale, shared_gn_bias,
           expert_w, expert_gn_scale, expert_gn_bias):
    x3 = x.reshape(_B, _C, _HW)

    ridx = jnp.arange(_T) // _W
    widx = jnp.arange(_T) % _W
    pcol = (ridx // _PS) * _PW + widx // _PS
    pmat = ((pcol[:, None] == jnp.arange(_PPC)[None, :])
            .astype(jnp.float32) / (_PS * _PS))

    dw9 = router_dw_w.reshape(_C, 9)
    pw8 = jnp.zeros((8, _C), jnp.float32).at[:_R].set(router_pw_w)
    fc8 = jnp.zeros((_E, 8), jnp.float32).at[:, :_R].set(router_fc_w)
    fcb = router_fc_b.reshape(_E, 1)
    expsb = jnp.stack([expert_gn_scale, expert_gn_bias], axis=-1)
    shsb = jnp.stack([shared_gn_scale, shared_gn_bias], axis=-1)

    out3 = pl.pallas_call(
        _fused_kernel,
        grid=(_B + 1, _NT),
        in_specs=[
            pl.BlockSpec((1, _C, _T),
                         lambda i, t: (jnp.minimum(i, _B - 1), 0,
                                       jnp.where(i >= _B, _NT - 1, t))),
            pl.BlockSpec((_T, _PPC), lambda i, t: (0, 0)),
            pl.BlockSpec((_C, 9), lambda i, t: (0, 0)),
            pl.BlockSpec((8, _C), lambda i, t: (0, 0)),
            pl.BlockSpec((_E, 8), lambda i, t: (0, 0)),
            pl.BlockSpec((_E, 1), lambda i, t: (0, 0)),
            pl.BlockSpec((_O, _C), lambda i, t: (0, 0)),
            pl.BlockSpec((_O, 2), lambda i, t: (0, 0)),
            pl.BlockSpec((_E, _O, _C), lambda i, t: (0, 0, 0)),
            pl.BlockSpec((_E, _O, 2), lambda i, t: (0, 0, 0)),
        ],
        out_specs=pl.BlockSpec((1, _O, _T),
                               lambda i, t: (jnp.maximum(i - 1, 0), 0,
                                             jnp.where(i == 0, 0, t))),
        out_shape=jax.ShapeDtypeStruct((_B, _O, _HW), jnp.float32),
        scratch_shapes=[
            pltpu.VMEM((2, _NT, _C, _T), jnp.bfloat16),
            pltpu.VMEM((2, _C, _C), jnp.float32),
            pltpu.VMEM((2, _NT, _C, _PPC), jnp.float32),
            pltpu.VMEM((3 * _O, _C), jnp.bfloat16),
            pltpu.VMEM((3 * _O, 8), jnp.float32),
        ],
    )(x3, pmat, dw9, pw8, fc8, fcb, shared_w, shsb, expert_w, expsb)

    return out3.reshape(_B, _O, _H, _W)


# speedup vs baseline: 1.1073x; 1.0837x over previous
import jax
import jax.numpy as jnp
from jax import lax
from jax.experimental import pallas as pl
from jax.experimental.pallas import tpu as pltpu

_B, _C, _O, _H, _W = 4, 96, 96, 224, 224
_E, _K, _PS, _NG = 8, 2, 8, 8
_R = 6
_HW = _H * _W
_THRESH = 0.01
_GS = _O // _NG
_T = 7168
_NT = _HW // _T
_CHUNK = 3584
_PW = _W // _PS
_PR = 2
_NP = _HW // _CHUNK


def _silu(v):
    return v * jax.nn.sigmoid(v)


def _prep_kernel(xa_ref, xb_ref, pmat_ref, dw_ref, pw_ref, fc_ref, fcb_ref,
                 shw_ref, shsb_ref, expw_ref, expsb_ref, mcat_ref, aff_ref):
    pm = pmat_ref[...]
    parts = []
    g = None
    for half_ref in (xa_ref, xb_ref):
        xm_h = half_ref[0]
        xmb = xm_h.astype(jnp.bfloat16)
        gh = lax.dot_general(xmb, xmb, (((1,), (1,)), ((), ())),
                             preferred_element_type=jnp.float32)
        g = gh if g is None else g + gh
        for k in range(_NP // 2):
            sub = xm_h[:, k * _CHUNK:(k + 1) * _CHUNK]
            parts.append(jnp.dot(sub, pm, preferred_element_type=jnp.float32))
    xm = jnp.concatenate(parts, axis=1)

    z32 = jnp.zeros((_C, 32), jnp.float32)
    xbig = jnp.concatenate([z32, xm, z32], axis=1)
    jcol = lax.broadcasted_iota(jnp.int32, (_C, _PW * _PW), 1) % _PW
    acc = jnp.zeros((_C, _PW * _PW), jnp.float32)
    for di in (-1, 0, 1):
        for dj in (-1, 0, 1):
            k9 = (di + 1) * 3 + (dj + 1)
            base = 32 + _PW * di + dj
            term = xbig[:, base:base + _PW * _PW] * dw_ref[:, k9:k9 + 1]
            if dj == -1:
                term = jnp.where(jcol == 0, 0.0, term)
            elif dj == 1:
                term = jnp.where(jcol == _PW - 1, 0.0, term)
            acc = acc + term
    xd = _silu(acc)
    xr = _silu(jnp.dot(pw_ref[...], xd, preferred_element_type=jnp.float32))
    gap = jnp.mean(xr, axis=1, keepdims=True)
    logits = jnp.dot(fc_ref[...], gap,
                     preferred_element_type=jnp.float32) + fcb_ref[...]
    io = lax.broadcasted_iota(jnp.int32, (_E, 1), 0)
    m1 = jnp.max(logits)
    i1 = jnp.min(jnp.where(logits == m1, io, _E))
    m2 = jnp.max(jnp.where(io == i1, -1e30, logits))
    i2 = jnp.min(jnp.where((logits == m2) & (io != i1), io, _E))
    e = jnp.exp(m2 - m1)
    w1 = 1.0 / (1.0 + e)
    w2 = e / (1.0 + e)
    w1 = jnp.where(w1 >= _THRESH, w1, 0.0)
    w2 = jnp.where(w2 >= _THRESH, w2, 0.0)

    s = jnp.sum(xm, axis=1, keepdims=True) * (_PS * _PS)
    gi = lax.broadcasted_iota(jnp.int32, (_O, _O), 0) // _GS
    gj = lax.broadcasted_iota(jnp.int32, (_O, _O), 1) // _GS
    pg = jnp.where(gi == gj, 1.0 / _GS, 0.0)
    for j, (idx, cw) in enumerate(((None, None), (i1, w1), (i2, w2))):
        if j == 0:
            wu = shw_ref[...]
            sb = shsb_ref[...]
            cval = jnp.float32(1.0)
        else:
            wu = expw_ref[idx]
            sb = expsb_ref[idx]
            cval = cw
        m = jnp.dot(wu, s, preferred_element_type=jnp.float32) / _HW
        t = jnp.dot(wu, g, preferred_element_type=jnp.float32)
        q = jnp.sum(t * wu, axis=1, keepdims=True) / _HW
        mu = jnp.dot(pg, m, preferred_element_type=jnp.float32)
        var = jnp.dot(pg, q, preferred_element_type=jnp.float32) - mu * mu
        rsig = lax.rsqrt(var + 1e-5)
        a = rsig * sb[:, 0:1]
        bv = sb[:, 1:2] - mu * a
        cc = jnp.zeros((_O, 1), jnp.float32) + cval
        mcat_ref[0, j * _O:(j + 1) * _O] = (wu * a).astype(jnp.bfloat16)
        aff_ref[0, j * _O:(j + 1) * _O] = jnp.concatenate(
            [bv, cc, jnp.zeros((_O, 6), jnp.float32)], axis=1)


def _main_kernel(x_ref, mcat_ref, aff_ref, out_ref):
    xt = x_ref[0]
    z = jnp.dot(mcat_ref[0], xt.astype(jnp.bfloat16),
                preferred_element_type=jnp.float32)
    acc = None
    for j in range(3):
        zj = z[_O * j:_O * (j + 1)]
        af = aff_ref[0, j * _O:(j + 1) * _O]
        tj = zj + af[:, 0:1]
        oj = af[:, 1:2] * (tj * jax.nn.sigmoid(tj))
        acc = oj if acc is None else acc + oj
    out_ref[0] = acc


def kernel(x, router_dw_w, router_pw_w, router_fc_w, router_fc_b,
           shared_w, shared_gn_scale, shared_gn_bias,
           expert_w, expert_gn_scale, expert_gn_bias):
    x3 = x.reshape(_B, _C, _HW)

    ridx = jnp.arange(_CHUNK) // _W
    widx = jnp.arange(_CHUNK) % _W
    pcol = (ridx // _PS) * _PW + widx // _PS
    pmat = ((pcol[:, None] == jnp.arange(_PR * _PW)[None, :])
            .astype(jnp.float32) / (_PS * _PS))

    dw9 = router_dw_w.reshape(_C, 9)
    pw8 = jnp.zeros((8, _C), jnp.float32).at[:_R].set(router_pw_w)
    fc8 = jnp.zeros((_E, 8), jnp.float32).at[:, :_R].set(router_fc_w)
    fcb = router_fc_b.reshape(_E, 1)
    expsb = jnp.stack([expert_gn_scale, expert_gn_bias], axis=-1)
    shsb = jnp.stack([shared_gn_scale, shared_gn_bias], axis=-1)

    mcat, aff = pl.pallas_call(
        _prep_kernel,
        grid=(_B,),
        in_specs=[
            pl.BlockSpec((1, _C, _HW // 2), lambda b: (b, 0, 0)),
            pl.BlockSpec((1, _C, _HW // 2), lambda b: (b, 0, 1)),
            pl.BlockSpec((_CHUNK, _PR * _PW), lambda b: (0, 0)),
            pl.BlockSpec((_C, 9), lambda b: (0, 0)),
            pl.BlockSpec((8, _C), lambda b: (0, 0)),
            pl.BlockSpec((_E, 8), lambda b: (0, 0)),
            pl.BlockSpec((_E, 1), lambda b: (0, 0)),
            pl.BlockSpec((_O, _C), lambda b: (0, 0)),
            pl.BlockSpec((_O, 2), lambda b: (0, 0)),
            pl.BlockSpec((_E, _O, _C), lambda b: (0, 0, 0)),
            pl.BlockSpec((_E, _O, 2), lambda b: (0, 0, 0)),
        ],
        out_specs=[pl.BlockSpec((1, 3 * _O, _C), lambda b: (b, 0, 0)),
                   pl.BlockSpec((1, 3 * _O, 8), lambda b: (b, 0, 0))],
        out_shape=[jax.ShapeDtypeStruct((_B, 3 * _O, _C), jnp.bfloat16),
                   jax.ShapeDtypeStruct((_B, 3 * _O, 8), jnp.float32)],
        compiler_params=pltpu.CompilerParams(
            dimension_semantics=("arbitrary",)),
    )(x3, x3, pmat, dw9, pw8, fc8, fcb, shared_w, shsb, expert_w, expsb)

    out3 = pl.pallas_call(
        _main_kernel,
        grid=(_B, _NT),
        in_specs=[pl.BlockSpec((1, _C, _T), lambda b, t: (b, 0, t)),
                  pl.BlockSpec((1, 3 * _O, _C), lambda b, t: (b, 0, 0)),
                  pl.BlockSpec((1, 3 * _O, 8), lambda b, t: (b, 0, 0))],
        out_specs=pl.BlockSpec((1, _O, _T), lambda b, t: (b, 0, t)),
        out_shape=jax.ShapeDtypeStruct((_B, _O, _HW), jnp.float32),
        compiler_params=pltpu.CompilerParams(
            dimension_semantics=("parallel", "arbitrary")),
    )(x3, mcat, aff)

    return out3.reshape(_B, _O, _H, _W)


# speedup vs baseline: 1.1093x; 1.0018x over previous
import jax
import jax.numpy as jnp
from jax import lax
from jax.experimental import pallas as pl
from jax.experimental.pallas import tpu as pltpu

_B, _C, _O, _H, _W = 4, 96, 96, 224, 224
_E, _K, _PS, _NG = 8, 2, 8, 8
_R = 6
_HW = _H * _W
_THRESH = 0.01
_GS = _O // _NG
_T = 12544
_NT = _HW // _T
_CHUNK = 3584
_PW = _W // _PS
_PR = 2
_NP = _HW // _CHUNK


def _silu(v):
    return v * jax.nn.sigmoid(v)


def _prep_kernel(xa_ref, xb_ref, pmat_ref, dw_ref, pw_ref, fc_ref, fcb_ref,
                 shw_ref, shsb_ref, expw_ref, expsb_ref, mcat_ref, aff_ref):
    pm = pmat_ref[...]
    parts = []
    g = None
    for half_ref in (xa_ref, xb_ref):
        xm_h = half_ref[0]
        xmb = xm_h.astype(jnp.bfloat16)
        gh = lax.dot_general(xmb, xmb, (((1,), (1,)), ((), ())),
                             preferred_element_type=jnp.float32)
        g = gh if g is None else g + gh
        for k in range(_NP // 2):
            sub = xm_h[:, k * _CHUNK:(k + 1) * _CHUNK]
            parts.append(jnp.dot(sub, pm, preferred_element_type=jnp.float32))
    xm = jnp.concatenate(parts, axis=1)

    z32 = jnp.zeros((_C, 32), jnp.float32)
    xbig = jnp.concatenate([z32, xm, z32], axis=1)
    jcol = lax.broadcasted_iota(jnp.int32, (_C, _PW * _PW), 1) % _PW
    acc = jnp.zeros((_C, _PW * _PW), jnp.float32)
    for di in (-1, 0, 1):
        for dj in (-1, 0, 1):
            k9 = (di + 1) * 3 + (dj + 1)
            base = 32 + _PW * di + dj
            term = xbig[:, base:base + _PW * _PW] * dw_ref[:, k9:k9 + 1]
            if dj == -1:
                term = jnp.where(jcol == 0, 0.0, term)
            elif dj == 1:
                term = jnp.where(jcol == _PW - 1, 0.0, term)
            acc = acc + term
    xd = _silu(acc)
    xr = _silu(jnp.dot(pw_ref[...], xd, preferred_element_type=jnp.float32))
    gap = jnp.mean(xr, axis=1, keepdims=True)
    logits = jnp.dot(fc_ref[...], gap,
                     preferred_element_type=jnp.float32) + fcb_ref[...]
    io = lax.broadcasted_iota(jnp.int32, (_E, 1), 0)
    m1 = jnp.max(logits)
    i1 = jnp.min(jnp.where(logits == m1, io, _E))
    m2 = jnp.max(jnp.where(io == i1, -1e30, logits))
    i2 = jnp.min(jnp.where((logits == m2) & (io != i1), io, _E))
    e = jnp.exp(m2 - m1)
    w1 = 1.0 / (1.0 + e)
    w2 = e / (1.0 + e)
    w1 = jnp.where(w1 >= _THRESH, w1, 0.0)
    w2 = jnp.where(w2 >= _THRESH, w2, 0.0)

    s = jnp.sum(xm, axis=1, keepdims=True) * (_PS * _PS)
    gi = lax.broadcasted_iota(jnp.int32, (_O, _O), 0) // _GS
    gj = lax.broadcasted_iota(jnp.int32, (_O, _O), 1) // _GS
    pg = jnp.where(gi == gj, 1.0 / _GS, 0.0)
    for j, (idx, cw) in enumerate(((None, None), (i1, w1), (i2, w2))):
        if j == 0:
            wu = shw_ref[...]
            sb = shsb_ref[...]
            cval = jnp.float32(1.0)
        else:
            wu = expw_ref[idx]
            sb = expsb_ref[idx]
            cval = cw
        m = jnp.dot(wu, s, preferred_element_type=jnp.float32) / _HW
        t = jnp.dot(wu, g, preferred_element_type=jnp.float32)
        q = jnp.sum(t * wu, axis=1, keepdims=True) / _HW
        mu = jnp.dot(pg, m, preferred_element_type=jnp.float32)
        var = jnp.dot(pg, q, preferred_element_type=jnp.float32) - mu * mu
        rsig = lax.rsqrt(var + 1e-5)
        a = rsig * sb[:, 0:1]
        bv = sb[:, 1:2] - mu * a
        cc = jnp.zeros((_O, 1), jnp.float32) + cval
        mcat_ref[0, j * _O:(j + 1) * _O] = (wu * a).astype(jnp.bfloat16)
        aff_ref[0, j * _O:(j + 1) * _O] = jnp.concatenate(
            [bv, cc, jnp.zeros((_O, 6), jnp.float32)], axis=1)


def _main_kernel(x_ref, mcat_ref, aff_ref, out_ref):
    xt = x_ref[0]
    z = jnp.dot(mcat_ref[0], xt.astype(jnp.bfloat16),
                preferred_element_type=jnp.float32)
    acc = None
    for j in range(3):
        zj = z[_O * j:_O * (j + 1)]
        af = aff_ref[0, j * _O:(j + 1) * _O]
        tj = zj + af[:, 0:1]
        oj = af[:, 1:2] * (tj * jax.nn.sigmoid(tj))
        acc = oj if acc is None else acc + oj
    out_ref[0] = acc


def kernel(x, router_dw_w, router_pw_w, router_fc_w, router_fc_b,
           shared_w, shared_gn_scale, shared_gn_bias,
           expert_w, expert_gn_scale, expert_gn_bias):
    x3 = x.reshape(_B, _C, _HW)

    ridx = jnp.arange(_CHUNK) // _W
    widx = jnp.arange(_CHUNK) % _W
    pcol = (ridx // _PS) * _PW + widx // _PS
    pmat = ((pcol[:, None] == jnp.arange(_PR * _PW)[None, :])
            .astype(jnp.float32) / (_PS * _PS))

    dw9 = router_dw_w.reshape(_C, 9)
    pw8 = jnp.zeros((8, _C), jnp.float32).at[:_R].set(router_pw_w)
    fc8 = jnp.zeros((_E, 8), jnp.float32).at[:, :_R].set(router_fc_w)
    fcb = router_fc_b.reshape(_E, 1)
    expsb = jnp.stack([expert_gn_scale, expert_gn_bias], axis=-1)
    shsb = jnp.stack([shared_gn_scale, shared_gn_bias], axis=-1)

    mcat, aff = pl.pallas_call(
        _prep_kernel,
        grid=(_B,),
        in_specs=[
            pl.BlockSpec((1, _C, _HW // 2), lambda b: (b, 0, 0)),
            pl.BlockSpec((1, _C, _HW // 2), lambda b: (b, 0, 1)),
            pl.BlockSpec((_CHUNK, _PR * _PW), lambda b: (0, 0)),
            pl.BlockSpec((_C, 9), lambda b: (0, 0)),
            pl.BlockSpec((8, _C), lambda b: (0, 0)),
            pl.BlockSpec((_E, 8), lambda b: (0, 0)),
            pl.BlockSpec((_E, 1), lambda b: (0, 0)),
            pl.BlockSpec((_O, _C), lambda b: (0, 0)),
            pl.BlockSpec((_O, 2), lambda b: (0, 0)),
            pl.BlockSpec((_E, _O, _C), lambda b: (0, 0, 0)),
            pl.BlockSpec((_E, _O, 2), lambda b: (0, 0, 0)),
        ],
        out_specs=[pl.BlockSpec((1, 3 * _O, _C), lambda b: (b, 0, 0)),
                   pl.BlockSpec((1, 3 * _O, 8), lambda b: (b, 0, 0))],
        out_shape=[jax.ShapeDtypeStruct((_B, 3 * _O, _C), jnp.bfloat16),
                   jax.ShapeDtypeStruct((_B, 3 * _O, 8), jnp.float32)],
        compiler_params=pltpu.CompilerParams(
            dimension_semantics=("arbitrary",)),
    )(x3, x3, pmat, dw9, pw8, fc8, fcb, shared_w, shsb, expert_w, expsb)

    out3 = pl.pallas_call(
        _main_kernel,
        grid=(_B, _NT),
        in_specs=[pl.BlockSpec((1, _C, _T), lambda b, t: (b, 0, t)),
                  pl.BlockSpec((1, 3 * _O, _C), lambda b, t: (b, 0, 0)),
                  pl.BlockSpec((1, 3 * _O, 8), lambda b, t: (b, 0, 0))],
        out_specs=pl.BlockSpec((1, _O, _T), lambda b, t: (b, 0, t)),
        out_shape=jax.ShapeDtypeStruct((_B, _O, _HW), jnp.float32),
        compiler_params=pltpu.CompilerParams(
            dimension_semantics=("parallel", "arbitrary")),
    )(x3, mcat, aff)

    return out3.reshape(_B, _O, _H, _W)
